# Initial kernel scaffold; baseline (speedup 1.0000x reference)
import jax
import jax.numpy as jnp
from jax.experimental import pallas as pl


def kernel(x, edge_index, batch, W1, b1, W2, b2, W_out, b_out):
    raise NotImplementedError("write your pallas kernel here")


# speedup vs baseline: 21.3376x; 21.3376x over previous
import functools

import jax
import jax.numpy as jnp
from jax import lax
from jax.experimental import pallas as pl
from jax.experimental.pallas import tpu as pltpu
from jax.experimental.pallas import tpu_sc as plsc

_N = 10000
_D = 128
_G = 64
_NCORE = 2
_NSUB = 16
_NW = _NCORE * _NSUB
_CHUNK = 128
_N_ACC = 10112
_RPT = _N_ACC // _NSUB
_N_DEG = 10240
_RPTD = _N_DEG // _NSUB
_BLK = 1000


def _sc_mesh():
    return plsc.VectorSubcoreMesh(core_axis_name="c", subcore_axis_name="s")


@functools.lru_cache
def _sc_deg_kernel(cpt):
    @functools.partial(
        pl.kernel,
        out_type=jax.ShapeDtypeStruct((_NCORE * _N_DEG,), jnp.float32),
        mesh=_sc_mesh(),
        scratch_types=[
            pltpu.VMEM((cpt, 2, _CHUNK), jnp.int32),
            pltpu.VMEM((_CHUNK,), jnp.float32),
            pltpu.VMEM_SHARED((_N_DEG,), jnp.float32),
        ],
    )
    def deg_kernel(edges_hbm, zeros_hbm, out_hbm, idx_v, ones_v, deg_sh):
        c = lax.axis_index("c")
        s = lax.axis_index("s")
        wid = s * _NCORE + c
        for j in range(_CHUNK // 16):
            ones_v[pl.ds(j * 16, 16)] = jnp.full((16,), 1.0, jnp.float32)
        pltpu.sync_copy(zeros_hbm, deg_sh.at[pl.ds(s * _RPTD, _RPTD)])
        pltpu.sync_copy(edges_hbm.at[wid], idx_v)
        plsc.subcore_barrier()

        def body(j, carry):
            pltpu.sync_copy(ones_v, deg_sh.at[idx_v.at[j, 1]], add=True)
            return carry

        lax.fori_loop(0, cpt, body, 0, unroll=False)
        plsc.subcore_barrier()
        pltpu.sync_copy(deg_sh.at[pl.ds(s * _RPTD, _RPTD)],
                        out_hbm.at[pl.ds(c * _N_DEG + s * _RPTD, _RPTD)])

    return deg_kernel


@functools.lru_cache
def _sc_agg_kernel(cpt):
    @functools.partial(
        pl.kernel,
        out_type=jax.ShapeDtypeStruct((_NCORE, _N_ACC, _D), jnp.float32),
        mesh=_sc_mesh(),
        scratch_types=[
            pltpu.VMEM((cpt, 2, _CHUNK), jnp.int32),
            pltpu.VMEM((_CHUNK, _D), jnp.float32),
            pltpu.VMEM_SHARED((_N_ACC, _D), jnp.float32),
            pltpu.SemaphoreType.DMA,
        ],
    )
    def agg_kernel(h_hbm, edges_hbm, zeros_hbm, out_hbm,
                   idx_v, rows, acc_sh, sem):
        c = lax.axis_index("c")
        s = lax.axis_index("s")
        wid = s * _NCORE + c
        pltpu.sync_copy(zeros_hbm, acc_sh.at[pl.ds(s * _RPT, _RPT)])
        pltpu.sync_copy(edges_hbm.at[wid], idx_v)
        plsc.subcore_barrier()

        def body(j, carry):
            pltpu.async_copy(h_hbm.at[idx_v.at[j, 0]], rows, sem).wait()
            pltpu.sync_copy(rows, acc_sh.at[idx_v.at[j, 1]], add=True)
            return carry

        lax.fori_loop(0, cpt, body, 0, unroll=False)

        plsc.subcore_barrier()
        pltpu.sync_copy(acc_sh.at[pl.ds(s * _RPT, _RPT)],
                        out_hbm.at[c, pl.ds(s * _RPT, _RPT)])

    return agg_kernel


def _dis_from(degp_ref):
    deg = degp_ref[0] + degp_ref[1]
    return jnp.where(deg > 0.0, lax.rsqrt(deg), 0.0)


def _tc_prep_body(x_ref, w_ref, degp_ref, o_ref):
    dis = _dis_from(degp_ref)
    h = jnp.dot(x_ref[...], w_ref[...], preferred_element_type=jnp.float32)
    o_ref[...] = h * dis


def _tc_prep(x, w1, degp3):
    return pl.pallas_call(
        _tc_prep_body,
        grid=(_N // _BLK,),
        in_specs=[
            pl.BlockSpec((_BLK, _D), lambda i: (i, 0)),
            pl.BlockSpec((_D, _D), lambda i: (0, 0)),
            pl.BlockSpec((_NCORE, _BLK, 1), lambda i: (0, i, 0)),
        ],
        out_specs=pl.BlockSpec((_BLK, _D), lambda i: (i, 0)),
        out_shape=jax.ShapeDtypeStruct((_N, _D), jnp.float32),
    )(x, w1, degp3)


def _tc_mid_body(aggp_ref, degp_ref, b_ref, w_ref, o_ref):
    dis = _dis_from(degp_ref)
    agg = aggp_ref[0] + aggp_ref[1]
    h = jnp.maximum(agg * dis + b_ref[...], 0.0)
    o_ref[...] = jnp.dot(h, w_ref[...], preferred_element_type=jnp.float32) * dis


def _tc_mid(aggp, degp3, b1, w2):
    return pl.pallas_call(
        _tc_mid_body,
        grid=(_N // _BLK,),
        in_specs=[
            pl.BlockSpec((_NCORE, _BLK, _D), lambda i: (0, i, 0)),
            pl.BlockSpec((_NCORE, _BLK, 1), lambda i: (0, i, 0)),
            pl.BlockSpec((1, _D), lambda i: (0, 0)),
            pl.BlockSpec((_D, _D), lambda i: (0, 0)),
        ],
        out_specs=pl.BlockSpec((_BLK, _D), lambda i: (i, 0)),
        out_shape=jax.ShapeDtypeStruct((_N, _D), jnp.float32),
    )(aggp, degp3, b1, w2)


def _tc_final_body(aggp_ref, degp_ref, b_ref, batch_ref, wout_ref, bout_ref,
                   o_ref, acc_s, acc_c):
    i = pl.program_id(0)
    dis = _dis_from(degp_ref)
    agg = aggp_ref[0] + aggp_ref[1]
    h = jnp.maximum(agg * dis + b_ref[...], 0.0)
    seg = lax.broadcasted_iota(jnp.int32, (_BLK, _G), 1)
    onehot = (batch_ref[...] == seg).astype(jnp.float32)
    dnum = (((0,), (0,)), ((), ()))
    sums = lax.dot_general(onehot, h, dnum,
                           preferred_element_type=jnp.float32)
    cnts = lax.dot_general(onehot, jnp.ones((_BLK, _D), jnp.float32), dnum,
                           preferred_element_type=jnp.float32)

    @pl.when(i == 0)
    def _():
        acc_s[...] = sums
        acc_c[...] = cnts

    @pl.when(i > 0)
    def _():
        acc_s[...] = acc_s[...] + sums
        acc_c[...] = acc_c[...] + cnts

    cnt = jnp.maximum(acc_c[...][:, :1], 1.0)
    proj = jnp.dot(acc_s[...], wout_ref[...],
                   preferred_element_type=jnp.float32)
    o_ref[...] = proj / cnt + bout_ref[...]


def _tc_final(aggp, degp3, b2, batch2, wout, bout):
    return pl.pallas_call(
        _tc_final_body,
        grid=(_N // _BLK,),
        in_specs=[
            pl.BlockSpec((_NCORE, _BLK, _D), lambda i: (0, i, 0)),
            pl.BlockSpec((_NCORE, _BLK, 1), lambda i: (0, i, 0)),
            pl.BlockSpec((1, _D), lambda i: (0, 0)),
            pl.BlockSpec((_BLK, 1), lambda i: (i, 0)),
            pl.BlockSpec((_D, 1), lambda i: (0, 0)),
            pl.BlockSpec((1, 1), lambda i: (0, 0)),
        ],
        out_specs=pl.BlockSpec((_G, 1), lambda i: (0, 0)),
        out_shape=jax.ShapeDtypeStruct((_G, 1), jnp.float32),
        scratch_shapes=[
            pltpu.VMEM((_G, _D), jnp.float32),
            pltpu.VMEM((_G, _D), jnp.float32),
        ],
    )(aggp, degp3, b2, batch2, wout, bout)


def kernel(x, edge_index, batch, W1, b1, W2, b2, W_out, b_out):
    e = edge_index.shape[1]
    e2 = e + _N
    cpt = -(-e2 // (_NW * _CHUNK))
    e2_pad = _NW * cpt * _CHUNK
    npad = e2_pad - e2

    loop = jnp.arange(_N, dtype=jnp.int32)
    pad_idx = jnp.arange(npad, dtype=jnp.int32)
    src = jnp.concatenate(
        [edge_index[0].astype(jnp.int32), loop, pad_idx % _N])
    dst = jnp.concatenate(
        [edge_index[1].astype(jnp.int32), loop, _N + pad_idx % (_N_ACC - _N)])
    edges4 = jnp.stack(
        [src.reshape(_NW, cpt, _CHUNK), dst.reshape(_NW, cpt, _CHUNK)], axis=2)

    zeros_deg = jnp.zeros((_RPTD,), jnp.float32)
    zeros_row = jnp.zeros((_RPT, _D), jnp.float32)

    degp = _sc_deg_kernel(cpt)(edges4, zeros_deg)
    degp3 = degp.reshape(_NCORE, _N_DEG, 1)

    hp1 = _tc_prep(x, W1, degp3)
    aggp1 = _sc_agg_kernel(cpt)(hp1, edges4, zeros_row)
    hp2 = _tc_mid(aggp1, degp3, b1.reshape(1, _D), W2)
    aggp2 = _sc_agg_kernel(cpt)(hp2, edges4, zeros_row)
    out = _tc_final(aggp2, degp3, b2.reshape(1, _D),
                    batch.reshape(_N, 1).astype(jnp.int32),
                    W_out, b_out.reshape(1, 1))
    return out.reshape(-1)


# speedup vs baseline: 26.3406x; 1.2345x over previous
import functools

import jax
import jax.numpy as jnp
from jax import lax
from jax.experimental import pallas as pl
from jax.experimental.pallas import tpu as pltpu
from jax.experimental.pallas import tpu_sc as plsc

_N = 10000
_D = 128
_G = 64
_NCORE = 2
_NSUB = 16
_NW = _NCORE * _NSUB
_CHUNK = 128
_N_ACC = 10112
_RPT = _N_ACC // _NSUB
_N_DEG = 10240
_RPTD = _N_DEG // _NSUB
_BLK = 1000


def _sc_mesh():
    return plsc.VectorSubcoreMesh(core_axis_name="c", subcore_axis_name="s")


@functools.lru_cache
def _sc_deg_kernel(cpt):
    @functools.partial(
        pl.kernel,
        out_type=jax.ShapeDtypeStruct((_NCORE * _N_DEG,), jnp.float32),
        mesh=_sc_mesh(),
        scratch_types=[
            pltpu.VMEM((cpt, 2, _CHUNK), jnp.int32),
            pltpu.VMEM((_CHUNK,), jnp.float32),
            pltpu.VMEM_SHARED((_N_DEG,), jnp.float32),
        ],
    )
    def deg_kernel(edges_hbm, zeros_hbm, out_hbm, idx_v, ones_v, deg_sh):
        c = lax.axis_index("c")
        s = lax.axis_index("s")
        wid = s * _NCORE + c
        for j in range(_CHUNK // 16):
            ones_v[pl.ds(j * 16, 16)] = jnp.full((16,), 1.0, jnp.float32)
        pltpu.sync_copy(zeros_hbm, deg_sh.at[pl.ds(s * _RPTD, _RPTD)])
        pltpu.sync_copy(edges_hbm.at[wid], idx_v)
        plsc.subcore_barrier()

        def body(j, carry):
            pltpu.sync_copy(ones_v, deg_sh.at[idx_v.at[j, 1]], add=True)
            return carry

        lax.fori_loop(0, cpt, body, 0, unroll=False)
        plsc.subcore_barrier()
        pltpu.sync_copy(deg_sh.at[pl.ds(s * _RPTD, _RPTD)],
                        out_hbm.at[pl.ds(c * _N_DEG + s * _RPTD, _RPTD)])

    return deg_kernel


_IDXB = 42


def _agg_block_sizes(cpt):
    sizes = []
    rem = cpt
    while rem > 0:
        b = min(_IDXB, rem)
        if rem - b == 2:
            b -= 2
        sizes.append(b)
        rem -= b
    return sizes


@functools.lru_cache
def _sc_agg_kernel(cpt):
    sizes = _agg_block_sizes(cpt)
    pipelined = (cpt % 2 == 0) and all(b >= 4 and b % 2 == 0 for b in sizes)
    idx_rows = sizes[0] if pipelined else cpt

    @functools.partial(
        pl.kernel,
        out_type=jax.ShapeDtypeStruct((_NCORE, _N_ACC, _D), jnp.float32),
        mesh=_sc_mesh(),
        scratch_types=[
            pltpu.VMEM((idx_rows, 2, _CHUNK), jnp.int32),
            pltpu.VMEM((_CHUNK, _D), jnp.float32),
            pltpu.VMEM((_CHUNK, _D), jnp.float32),
            pltpu.VMEM_SHARED((_N_ACC, _D), jnp.float32),
            pltpu.SemaphoreType.DMA,
            pltpu.SemaphoreType.DMA,
        ],
    )
    def agg_kernel(h_hbm, edges_hbm, zeros_hbm, out_hbm,
                   idx_v, rows0, rows1, acc_sh, gsem, ssem):
        c = lax.axis_index("c")
        s = lax.axis_index("s")
        wid = s * _NCORE + c
        pltpu.sync_copy(zeros_hbm, acc_sh.at[pl.ds(s * _RPT, _RPT)])
        plsc.subcore_barrier()

        def g(j, buf):
            pltpu.async_copy(h_hbm.at[idx_v.at[j, 0]], buf, gsem)

        def sc(j, buf):
            pltpu.async_copy(buf, acc_sh.at[idx_v.at[j, 1]], ssem, add=True)

        def wg():
            pltpu.make_async_copy(
                h_hbm.at[idx_v.at[0, 0]], rows0, gsem).wait()

        def ws():
            pltpu.make_async_copy(
                rows0, acc_sh.at[idx_v.at[0, 1]], ssem).wait()

        if pipelined:
            j0 = 0
            for bsz in sizes:
                pltpu.sync_copy(edges_hbm.at[wid, pl.ds(j0, bsz)],
                                idx_v.at[pl.ds(0, bsz)])
                g(0, rows0)
                wg(); g(1, rows1); sc(0, rows0)
                wg(); ws(); g(2, rows0); sc(1, rows1)

                def body(p, carry):
                    j = 2 * p
                    wg(); ws(); g(j + 1, rows1); sc(j, rows0)
                    wg(); ws(); g(j + 2, rows0); sc(j + 1, rows1)
                    return carry

                lax.fori_loop(1, bsz // 2 - 1, body, 0, unroll=False)
                j = bsz - 2
                wg(); ws(); g(j + 1, rows1); sc(j, rows0)
                wg(); ws(); sc(j + 1, rows1)
                ws()
                j0 += bsz
        else:
            pltpu.sync_copy(edges_hbm.at[wid], idx_v)

            def body(j, carry):
                pltpu.async_copy(h_hbm.at[idx_v.at[j, 0]], rows0, gsem).wait()
                pltpu.sync_copy(rows0, acc_sh.at[idx_v.at[j, 1]], add=True)
                return carry

            lax.fori_loop(0, cpt, body, 0, unroll=False)

        plsc.subcore_barrier()
        pltpu.sync_copy(acc_sh.at[pl.ds(s * _RPT, _RPT)],
                        out_hbm.at[c, pl.ds(s * _RPT, _RPT)])

    return agg_kernel


def _dis_from(degp_ref):
    deg = degp_ref[0] + degp_ref[1]
    return jnp.where(deg > 0.0, lax.rsqrt(deg), 0.0)


def _tc_prep_body(x_ref, w_ref, degp_ref, o_ref):
    dis = _dis_from(degp_ref)
    h = jnp.dot(x_ref[...], w_ref[...], preferred_element_type=jnp.float32)
    o_ref[...] = h * dis


def _tc_prep(x, w1, degp3):
    return pl.pallas_call(
        _tc_prep_body,
        grid=(_N // _BLK,),
        in_specs=[
            pl.BlockSpec((_BLK, _D), lambda i: (i, 0)),
            pl.BlockSpec((_D, _D), lambda i: (0, 0)),
            pl.BlockSpec((_NCORE, _BLK, 1), lambda i: (0, i, 0)),
        ],
        out_specs=pl.BlockSpec((_BLK, _D), lambda i: (i, 0)),
        out_shape=jax.ShapeDtypeStruct((_N, _D), jnp.float32),
    )(x, w1, degp3)


def _tc_mid_body(aggp_ref, degp_ref, b_ref, w_ref, o_ref):
    dis = _dis_from(degp_ref)
    agg = aggp_ref[0] + aggp_ref[1]
    h = jnp.maximum(agg * dis + b_ref[...], 0.0)
    o_ref[...] = jnp.dot(h, w_ref[...], preferred_element_type=jnp.float32) * dis


def _tc_mid(aggp, degp3, b1, w2):
    return pl.pallas_call(
        _tc_mid_body,
        grid=(_N // _BLK,),
        in_specs=[
            pl.BlockSpec((_NCORE, _BLK, _D), lambda i: (0, i, 0)),
            pl.BlockSpec((_NCORE, _BLK, 1), lambda i: (0, i, 0)),
            pl.BlockSpec((1, _D), lambda i: (0, 0)),
            pl.BlockSpec((_D, _D), lambda i: (0, 0)),
        ],
        out_specs=pl.BlockSpec((_BLK, _D), lambda i: (i, 0)),
        out_shape=jax.ShapeDtypeStruct((_N, _D), jnp.float32),
    )(aggp, degp3, b1, w2)


def _tc_final_body(aggp_ref, degp_ref, b_ref, batch_ref, wout_ref, bout_ref,
                   o_ref, acc_s, acc_c):
    i = pl.program_id(0)
    dis = _dis_from(degp_ref)
    agg = aggp_ref[0] + aggp_ref[1]
    h = jnp.maximum(agg * dis + b_ref[...], 0.0)
    seg = lax.broadcasted_iota(jnp.int32, (_BLK, _G), 1)
    onehot = (batch_ref[...] == seg).astype(jnp.float32)
    dnum = (((0,), (0,)), ((), ()))
    sums = lax.dot_general(onehot, h, dnum,
                           preferred_element_type=jnp.float32)
    cnts = lax.dot_general(onehot, jnp.ones((_BLK, _D), jnp.float32), dnum,
                           preferred_element_type=jnp.float32)

    @pl.when(i == 0)
    def _():
        acc_s[...] = sums
        acc_c[...] = cnts

    @pl.when(i > 0)
    def _():
        acc_s[...] = acc_s[...] + sums
        acc_c[...] = acc_c[...] + cnts

    cnt = jnp.maximum(acc_c[...][:, :1], 1.0)
    proj = jnp.dot(acc_s[...], wout_ref[...],
                   preferred_element_type=jnp.float32)
    o_ref[...] = proj / cnt + bout_ref[...]


def _tc_final(aggp, degp3, b2, batch2, wout, bout):
    return pl.pallas_call(
        _tc_final_body,
        grid=(_N // _BLK,),
        in_specs=[
            pl.BlockSpec((_NCORE, _BLK, _D), lambda i: (0, i, 0)),
            pl.BlockSpec((_NCORE, _BLK, 1), lambda i: (0, i, 0)),
            pl.BlockSpec((1, _D), lambda i: (0, 0)),
            pl.BlockSpec((_BLK, 1), lambda i: (i, 0)),
            pl.BlockSpec((_D, 1), lambda i: (0, 0)),
            pl.BlockSpec((1, 1), lambda i: (0, 0)),
        ],
        out_specs=pl.BlockSpec((_G, 1), lambda i: (0, 0)),
        out_shape=jax.ShapeDtypeStruct((_G, 1), jnp.float32),
        scratch_shapes=[
            pltpu.VMEM((_G, _D), jnp.float32),
            pltpu.VMEM((_G, _D), jnp.float32),
        ],
    )(aggp, degp3, b2, batch2, wout, bout)


def kernel(x, edge_index, batch, W1, b1, W2, b2, W_out, b_out):
    e = edge_index.shape[1]
    e2 = e + _N
    cpt = -(-e2 // (_NW * _CHUNK))
    cpt += cpt % 2
    e2_pad = _NW * cpt * _CHUNK
    npad = e2_pad - e2

    loop = jnp.arange(_N, dtype=jnp.int32)
    pad_idx = jnp.arange(npad, dtype=jnp.int32)
    src = jnp.concatenate(
        [edge_index[0].astype(jnp.int32), loop, pad_idx % _N])
    dst = jnp.concatenate(
        [edge_index[1].astype(jnp.int32), loop, _N + pad_idx % (_N_ACC - _N)])
    edges4 = jnp.stack(
        [src.reshape(_NW, cpt, _CHUNK), dst.reshape(_NW, cpt, _CHUNK)], axis=2)

    zeros_deg = jnp.zeros((_RPTD,), jnp.float32)
    zeros_row = jnp.zeros((_RPT, _D), jnp.float32)

    degp = _sc_deg_kernel(cpt)(edges4, zeros_deg)
    degp3 = degp.reshape(_NCORE, _N_DEG, 1)

    hp1 = _tc_prep(x, W1, degp3)
    aggp1 = _sc_agg_kernel(cpt)(hp1, edges4, zeros_row)
    hp2 = _tc_mid(aggp1, degp3, b1.reshape(1, _D), W2)
    aggp2 = _sc_agg_kernel(cpt)(hp2, edges4, zeros_row)
    out = _tc_final(aggp2, degp3, b2.reshape(1, _D),
                    batch.reshape(_N, 1).astype(jnp.int32),
                    W_out, b_out.reshape(1, 1))
    return out.reshape(-1)


# speedup vs baseline: 28.8629x; 1.0958x over previous
import functools

import jax
import jax.numpy as jnp
from jax import lax
from jax.experimental import pallas as pl
from jax.experimental.pallas import tpu as pltpu
from jax.experimental.pallas import tpu_sc as plsc

_N = 10000
_D = 128
_G = 64
_NCORE = 2
_NSUB = 16
_NW = _NCORE * _NSUB
_CHUNK = 96
_N_ACC = 10112
_RPT = _N_ACC // _NSUB
_N_DEG = 10240
_RPTD = _N_DEG // _NSUB
_BLK = 1000


def _sc_mesh():
    return plsc.VectorSubcoreMesh(core_axis_name="c", subcore_axis_name="s")


@functools.lru_cache
def _sc_deg_kernel(cpt):
    @functools.partial(
        pl.kernel,
        out_type=jax.ShapeDtypeStruct((_NCORE * _N_DEG,), jnp.float32),
        mesh=_sc_mesh(),
        scratch_types=[
            pltpu.VMEM((cpt, 2, _CHUNK), jnp.int32),
            pltpu.VMEM((_CHUNK,), jnp.float32),
            pltpu.VMEM_SHARED((_N_DEG,), jnp.float32),
        ],
    )
    def deg_kernel(edges_hbm, zeros_hbm, out_hbm, idx_v, ones_v, deg_sh):
        c = lax.axis_index("c")
        s = lax.axis_index("s")
        wid = s * _NCORE + c
        for j in range(_CHUNK // 16):
            ones_v[pl.ds(j * 16, 16)] = jnp.full((16,), 1.0, jnp.float32)
        pltpu.sync_copy(zeros_hbm, deg_sh.at[pl.ds(s * _RPTD, _RPTD)])
        pltpu.sync_copy(edges_hbm.at[wid], idx_v)
        plsc.subcore_barrier()

        def body(j, carry):
            pltpu.sync_copy(ones_v, deg_sh.at[idx_v.at[j, 1]], add=True)
            return carry

        lax.fori_loop(0, cpt, body, 0, unroll=False)
        plsc.subcore_barrier()
        pltpu.sync_copy(deg_sh.at[pl.ds(s * _RPTD, _RPTD)],
                        out_hbm.at[pl.ds(c * _N_DEG + s * _RPTD, _RPTD)])

    return deg_kernel


_IDXB = 42


def _agg_block_sizes(cpt):
    sizes = []
    rem = cpt
    while rem > 0:
        b = min(_IDXB, rem)
        sizes.append(b)
        rem -= b
    return sizes


@functools.lru_cache
def _sc_agg_kernel(cpt):
    sizes = _agg_block_sizes(cpt)
    pipelined = all(b >= 3 and b % 3 == 0 for b in sizes)
    idx_rows = sizes[0] if pipelined else cpt

    @functools.partial(
        pl.kernel,
        out_type=jax.ShapeDtypeStruct((_NCORE, _N_ACC, _D), jnp.float32),
        mesh=_sc_mesh(),
        scratch_types=[
            pltpu.VMEM((idx_rows, 2, _CHUNK), jnp.int32),
            pltpu.VMEM((_CHUNK, _D), jnp.float32),
            pltpu.VMEM((_CHUNK, _D), jnp.float32),
            pltpu.VMEM((_CHUNK, _D), jnp.float32),
            pltpu.VMEM_SHARED((_N_ACC, _D), jnp.float32),
            pltpu.SemaphoreType.DMA,
            pltpu.SemaphoreType.DMA,
            pltpu.SemaphoreType.DMA,
            pltpu.SemaphoreType.DMA,
            pltpu.SemaphoreType.DMA,
            pltpu.SemaphoreType.DMA,
        ],
    )
    def agg_kernel(h_hbm, edges_hbm, zeros_hbm, out_hbm,
                   idx_v, rows0, rows1, rows2, acc_sh,
                   g0, g1, g2, s0, s1, s2):
        c = lax.axis_index("c")
        s = lax.axis_index("s")
        wid = s * _NCORE + c
        pltpu.sync_copy(zeros_hbm, acc_sh.at[pl.ds(s * _RPT, _RPT)])
        plsc.subcore_barrier()

        bufs = (rows0, rows1, rows2)
        gsems = (g0, g1, g2)
        ssems = (s0, s1, s2)

        def g(j, k):
            pltpu.async_copy(h_hbm.at[idx_v.at[j, 0]], bufs[k], gsems[k])

        def sc(j, k):
            pltpu.async_copy(bufs[k], acc_sh.at[idx_v.at[j, 1]],
                             ssems[k], add=True)

        def wg(k):
            pltpu.make_async_copy(
                h_hbm.at[idx_v.at[0, 0]], bufs[k], gsems[k]).wait()

        def ws(k):
            pltpu.make_async_copy(
                bufs[k], acc_sh.at[idx_v.at[0, 1]], ssems[k]).wait()

        if pipelined:
            j0 = 0
            for bsz in sizes:
                pltpu.sync_copy(edges_hbm.at[wid, pl.ds(j0, bsz)],
                                idx_v.at[pl.ds(0, bsz)])
                g(0, 0)
                g(1, 1)
                wg(0); g(2, 2); sc(0, 0)

                def body(p, carry):
                    j = 3 * p + 1
                    wg(1); ws(0); g(j + 2, 0); sc(j, 1)
                    wg(2); ws(1); g(j + 3, 1); sc(j + 1, 2)
                    wg(0); ws(2); g(j + 4, 2); sc(j + 2, 0)
                    return carry

                lax.fori_loop(0, (bsz - 3) // 3, body, 0, unroll=False)
                wg(1); ws(0); sc(bsz - 2, 1)
                wg(2); ws(1); sc(bsz - 1, 2)
                ws(2)
                j0 += bsz
        else:
            pltpu.sync_copy(edges_hbm.at[wid], idx_v)

            def body(j, carry):
                pltpu.async_copy(
                    h_hbm.at[idx_v.at[j, 0]], rows0, g0).wait()
                pltpu.sync_copy(rows0, acc_sh.at[idx_v.at[j, 1]], add=True)
                return carry

            lax.fori_loop(0, cpt, body, 0, unroll=False)

        plsc.subcore_barrier()
        pltpu.sync_copy(acc_sh.at[pl.ds(s * _RPT, _RPT)],
                        out_hbm.at[c, pl.ds(s * _RPT, _RPT)])

    return agg_kernel


def _dis_from(degp_ref):
    deg = degp_ref[0] + degp_ref[1]
    return jnp.where(deg > 0.0, lax.rsqrt(deg), 0.0)


def _tc_prep_body(x_ref, w_ref, degp_ref, o_ref):
    dis = _dis_from(degp_ref)
    h = jnp.dot(x_ref[...], w_ref[...], preferred_element_type=jnp.float32,
                precision=lax.Precision.HIGHEST)
    o_ref[...] = h * dis


def _tc_prep(x, w1, degp3):
    return pl.pallas_call(
        _tc_prep_body,
        grid=(_N // _BLK,),
        in_specs=[
            pl.BlockSpec((_BLK, _D), lambda i: (i, 0)),
            pl.BlockSpec((_D, _D), lambda i: (0, 0)),
            pl.BlockSpec((_NCORE, _BLK, 1), lambda i: (0, i, 0)),
        ],
        out_specs=pl.BlockSpec((_BLK, _D), lambda i: (i, 0)),
        out_shape=jax.ShapeDtypeStruct((_N, _D), jnp.float32),
    )(x, w1, degp3)


def _tc_mid_body(aggp_ref, degp_ref, b_ref, w_ref, o_ref):
    dis = _dis_from(degp_ref)
    agg = aggp_ref[0] + aggp_ref[1]
    h = jnp.maximum(agg * dis + b_ref[...], 0.0)
    o_ref[...] = jnp.dot(h, w_ref[...], preferred_element_type=jnp.float32,
                         precision=lax.Precision.HIGHEST) * dis


def _tc_mid(aggp, degp3, b1, w2):
    return pl.pallas_call(
        _tc_mid_body,
        grid=(_N // _BLK,),
        in_specs=[
            pl.BlockSpec((_NCORE, _BLK, _D), lambda i: (0, i, 0)),
            pl.BlockSpec((_NCORE, _BLK, 1), lambda i: (0, i, 0)),
            pl.BlockSpec((1, _D), lambda i: (0, 0)),
            pl.BlockSpec((_D, _D), lambda i: (0, 0)),
        ],
        out_specs=pl.BlockSpec((_BLK, _D), lambda i: (i, 0)),
        out_shape=jax.ShapeDtypeStruct((_N, _D), jnp.float32),
    )(aggp, degp3, b1, w2)


def _tc_final_body(aggp_ref, degp_ref, b_ref, batch_ref, wout_ref, bout_ref,
                   o_ref, acc_s, acc_c):
    i = pl.program_id(0)
    dis = _dis_from(degp_ref)
    agg = aggp_ref[0] + aggp_ref[1]
    h = jnp.maximum(agg * dis + b_ref[...], 0.0)
    seg = lax.broadcasted_iota(jnp.int32, (_BLK, _G), 1)
    onehot = (batch_ref[...] == seg).astype(jnp.float32)
    dnum = (((0,), (0,)), ((), ()))
    sums = lax.dot_general(onehot, h, dnum,
                           preferred_element_type=jnp.float32,
                           precision=lax.Precision.HIGHEST)
    cnts = lax.dot_general(onehot, jnp.ones((_BLK, _D), jnp.float32), dnum,
                           preferred_element_type=jnp.float32,
                           precision=lax.Precision.HIGHEST)

    @pl.when(i == 0)
    def _():
        acc_s[...] = sums
        acc_c[...] = cnts

    @pl.when(i > 0)
    def _():
        acc_s[...] = acc_s[...] + sums
        acc_c[...] = acc_c[...] + cnts

    cnt = jnp.maximum(acc_c[...][:, :1], 1.0)
    pooled = acc_s[...] / cnt
    proj = jnp.dot(pooled, wout_ref[...],
                   preferred_element_type=jnp.float32,
                   precision=lax.Precision.HIGHEST)
    o_ref[...] = proj + bout_ref[...]


def _tc_final(aggp, degp3, b2, batch2, wout, bout):
    return pl.pallas_call(
        _tc_final_body,
        grid=(_N // _BLK,),
        in_specs=[
            pl.BlockSpec((_NCORE, _BLK, _D), lambda i: (0, i, 0)),
            pl.BlockSpec((_NCORE, _BLK, 1), lambda i: (0, i, 0)),
            pl.BlockSpec((1, _D), lambda i: (0, 0)),
            pl.BlockSpec((_BLK, 1), lambda i: (i, 0)),
            pl.BlockSpec((_D, 1), lambda i: (0, 0)),
            pl.BlockSpec((1, 1), lambda i: (0, 0)),
        ],
        out_specs=pl.BlockSpec((_G, 1), lambda i: (0, 0)),
        out_shape=jax.ShapeDtypeStruct((_G, 1), jnp.float32),
        scratch_shapes=[
            pltpu.VMEM((_G, _D), jnp.float32),
            pltpu.VMEM((_G, _D), jnp.float32),
        ],
    )(aggp, degp3, b2, batch2, wout, bout)


def kernel(x, edge_index, batch, W1, b1, W2, b2, W_out, b_out):
    e = edge_index.shape[1]
    e2 = e + _N
    cpt = -(-e2 // (_NW * _CHUNK))
    cpt += -cpt % 3
    e2_pad = _NW * cpt * _CHUNK
    npad = e2_pad - e2

    loop = jnp.arange(_N, dtype=jnp.int32)
    pad_idx = jnp.arange(npad, dtype=jnp.int32)
    src = jnp.concatenate(
        [edge_index[0].astype(jnp.int32), loop, pad_idx % _N])
    dst = jnp.concatenate(
        [edge_index[1].astype(jnp.int32), loop, _N + pad_idx % (_N_ACC - _N)])
    edges4 = jnp.stack(
        [src.reshape(_NW, cpt, _CHUNK), dst.reshape(_NW, cpt, _CHUNK)], axis=2)

    zeros_deg = jnp.zeros((_RPTD,), jnp.float32)
    zeros_row = jnp.zeros((_RPT, _D), jnp.float32)

    degp = _sc_deg_kernel(cpt)(edges4, zeros_deg)
    degp3 = degp.reshape(_NCORE, _N_DEG, 1)

    hp1 = _tc_prep(x, W1, degp3)
    aggp1 = _sc_agg_kernel(cpt)(hp1, edges4, zeros_row)
    hp2 = _tc_mid(aggp1, degp3, b1.reshape(1, _D), W2)
    aggp2 = _sc_agg_kernel(cpt)(hp2, edges4, zeros_row)
    out = _tc_final(aggp2, degp3, b2.reshape(1, _D),
                    batch.reshape(_N, 1).astype(jnp.int32),
                    W_out, b_out.reshape(1, 1))
    return out.reshape(-1)
